# Initial kernel scaffold; baseline (speedup 1.0000x reference)
import jax
import jax.numpy as jnp
from jax.experimental import pallas as pl


def kernel(x_instr, x_block, ei_flow, ei_in, ei_cont, Kw, Kb, Qw, Qb, Vw, Vb, Aw, Ab, skip, a_rel, m_rel, p_rel, ln_g, ln_b, wi_f, wh_f, bl_f, wi_b, wh_b, bl_b, att_w, att_b, pool_w, pool_b, w1, b1, w2, b2, w3, b3, w4, b4):
    raise NotImplementedError("write your pallas kernel here")


# speedup vs baseline: 8.8460x; 8.8460x over previous
import functools

import jax
import jax.numpy as jnp
from jax import lax
from jax.experimental import pallas as pl
from jax.experimental.pallas import tpu as pltpu
from jax.experimental.pallas import tpu_sc as plsc

H = 4
DH = 64
C = 256
N0 = 8000
N1 = 2000
NC = 2
NS = 16
NW = NC * NS
CH = 128

EF = 100000
EI = 30000
EC = 30000
EF_P = 102400
EI_P = 32768
EC_P = 32768
N0_P = 8192
N1_P = 2048

_SDS = jax.ShapeDtypeStruct
_f32 = jnp.float32


def _mesh():
    return plsc.VectorSubcoreMesh(core_axis_name="c", subcore_axis_name="s",
                                  num_cores=NC, num_subcores=NS)


@functools.lru_cache(maxsize=None)
def _sc_gather_fn(n, d, e_pad):
    b_per_w = e_pad // NW
    n_chunks = b_per_w // CH

    @functools.partial(
        pl.kernel,
        out_type=_SDS((e_pad, d), _f32),
        mesh=_mesh(),
        scratch_types=[
            pltpu.VMEM((CH,), jnp.int32),
            pltpu.VMEM((CH, d), _f32),
            pltpu.SemaphoreType.DMA,
        ],
    )
    def k(table_hbm, idx_hbm, out_hbm, idx_v, rows_v, sem):
        wid = lax.axis_index("s") * NC + lax.axis_index("c")

        def body(i, _):
            base = wid * b_per_w + i * CH
            pltpu.sync_copy(idx_hbm.at[pl.ds(base, CH)], idx_v)
            pltpu.async_copy(table_hbm.at[idx_v], rows_v, sem).wait()
            pltpu.sync_copy(rows_v, out_hbm.at[pl.ds(base, CH)])
            return 0

        lax.fori_loop(0, n_chunks, body, 0)

    return k


def _sc_gather(table, idx):
    n, d = table.shape
    (e_pad,) = idx.shape
    return _sc_gather_fn(n, d, e_pad)(table, idx)


@functools.lru_cache(maxsize=None)
def _sc_scatter_fn(e_pad, n_pad):
    per_tile = e_pad // NS
    n_chunks = per_tile // CH
    hn = n_pad // 2
    rs = hn // NS

    @functools.partial(
        pl.kernel,
        out_type=[_SDS((n_pad, 128), _f32), _SDS((n_pad, 128), _f32),
                  _SDS((NS, n_pad * 8), _f32)],
        mesh=_mesh(),
        scratch_types=[
            pltpu.VMEM((CH,), jnp.int32),
            pltpu.VMEM((CH,), jnp.int32),
            pltpu.VMEM((CH, 128), _f32),
            pltpu.VMEM((CH * 8 + 16,), _f32),
            pltpu.VMEM((n_pad * 8,), _f32),
            pltpu.VMEM_SHARED((hn + 8, 128), _f32),
        ],
    )
    def k(dst_hbm, msg0_hbm, msg1_hbm, ex_hbm, num0_hbm, num1_hbm, den_hbm,
          idx_v, idx2_v, rows_v, ex_v, den_l, accn_s):
        cid = lax.axis_index("c")
        sid = lax.axis_index("s")
        col = lax.broadcasted_iota(jnp.int32, (16,), 0)
        lane8 = col < 8
        z16 = jnp.zeros((16,), _f32)

        def zrow(r, _):
            for j in range(8):
                rows_v[r, j * 16:(j + 1) * 16] = z16
            return 0

        lax.fori_loop(0, CH, zrow, 0)

        def zden(r, _):
            den_l[pl.ds(r * 16, 16)] = z16
            return 0

        lax.fori_loop(0, n_pad // 2, zden, 0)

        for p in range(2):
            lo = p * hn
            for o in range(0, rs, CH):
                pltpu.sync_copy(rows_v.at[pl.ds(0, min(CH, rs - o))],
                                accn_s.at[pl.ds(sid * rs + o,
                                                min(CH, rs - o))])

            @pl.when(sid == 0)
            def _():
                pltpu.sync_copy(rows_v.at[pl.ds(0, 8)],
                                accn_s.at[pl.ds(hn, 8)])

            plsc.subcore_barrier()

            def body(i, _):
                base = sid * per_tile + i * CH
                pltpu.sync_copy(dst_hbm.at[pl.ds(base, CH)], idx_v)

                def remap(g, _):
                    dvec = idx_v[pl.ds(g * 16, 16)]
                    ok = (dvec >= lo) & (dvec < lo + hn)
                    idx2_v[pl.ds(g * 16, 16)] = jnp.where(ok, dvec - lo, hn)
                    return 0

                lax.fori_loop(0, CH // 16, remap, 0)

                @pl.when(cid == 0)
                def _():
                    pltpu.sync_copy(msg0_hbm.at[pl.ds(base, CH)], rows_v)
                    pltpu.sync_copy(rows_v, accn_s.at[idx2_v], add=True)
                    if p == 0:
                        pltpu.sync_copy(ex_hbm.at[pl.ds(base * 8, CH * 8)],
                                        ex_v.at[pl.ds(0, CH * 8)])

                        def dacc(g, _):
                            dvec = idx_v[pl.ds(g * 16, 16)]
                            for j in range(16):
                                ds = dvec[j]
                                vals = jnp.where(
                                    lane8,
                                    ex_v[pl.ds(g * 128 + j * 8, 16)], 0.0)
                                plsc.addupdate(
                                    den_l.at[pl.ds(ds * 8, 16)], vals)
                            return 0

                        lax.fori_loop(0, CH // 16, dacc, 0)

                @pl.when(cid == 1)
                def _():
                    pltpu.sync_copy(msg1_hbm.at[pl.ds(base, CH)], rows_v)
                    pltpu.sync_copy(rows_v, accn_s.at[idx2_v], add=True)

                return 0

            lax.fori_loop(0, n_chunks, body, 0)
            plsc.subcore_barrier()

            rbase = sid * rs

            @pl.when(cid == 0)
            def _():
                for o in range(0, rs, CH):
                    sz = min(CH, rs - o)
                    pltpu.sync_copy(accn_s.at[pl.ds(rbase + o, sz)],
                                    num0_hbm.at[pl.ds(lo + rbase + o, sz)])

            @pl.when(cid == 1)
            def _():
                for o in range(0, rs, CH):
                    sz = min(CH, rs - o)
                    pltpu.sync_copy(accn_s.at[pl.ds(rbase + o, sz)],
                                    num1_hbm.at[pl.ds(lo + rbase + o, sz)])

            plsc.subcore_barrier()

        @pl.when(cid == 0)
        def _():
            pltpu.sync_copy(den_l, den_hbm.at[sid])

    return k


def _sc_scatter(dst, msg0, msg1, ex, n_pad):
    (e_pad,) = dst.shape
    num0, num1, den = _sc_scatter_fn(e_pad, n_pad)(dst, msg0, msg1, ex)
    return num0, num1, den.reshape(NS, n_pad, 8)


@functools.lru_cache(maxsize=None)
def _proj_fn(n, n_rel):
    tn = 400
    grid = (n // tn,)

    def body(x_ref, kw_ref, kb_ref, qw_ref, qb_ref, vw_ref, vb_ref,
             ar_ref, mr_ref, q_out, *kavm_outs):
        x = x_ref[...]
        kk = jnp.dot(x, kw_ref[...], preferred_element_type=_f32) + kb_ref[...]
        qq = jnp.dot(x, qw_ref[...], preferred_element_type=_f32) + qb_ref[...]
        vv = jnp.dot(x, vw_ref[...], preferred_element_type=_f32) + vb_ref[...]
        q_out[...] = qq
        for r in range(n_rel):
            for h in range(H):
                ks = kk[:, h * DH:(h + 1) * DH]
                vs = vv[:, h * DH:(h + 1) * DH]
                kavm_outs[r][:, h * DH:(h + 1) * DH] = jnp.dot(
                    ks, ar_ref[r, h], preferred_element_type=_f32)
                kavm_outs[r][:, C + h * DH:C + (h + 1) * DH] = jnp.dot(
                    vs, mr_ref[r, h], preferred_element_type=_f32)

    full = lambda i: (0, 0)
    return pl.pallas_call(
        body,
        grid=grid,
        in_specs=[
            pl.BlockSpec((tn, C), lambda i: (i, 0)),
            pl.BlockSpec((C, C), full), pl.BlockSpec((1, C), full),
            pl.BlockSpec((C, C), full), pl.BlockSpec((1, C), full),
            pl.BlockSpec((C, C), full), pl.BlockSpec((1, C), full),
            pl.BlockSpec((n_rel, H, DH, DH), lambda i: (0, 0, 0, 0)),
            pl.BlockSpec((n_rel, H, DH, DH), lambda i: (0, 0, 0, 0)),
        ],
        out_specs=[pl.BlockSpec((tn, C), lambda i: (i, 0))] +
                  [pl.BlockSpec((tn, 2 * C), lambda i: (i, 0))] * n_rel,
        out_shape=[_SDS((n, C), _f32)] + [_SDS((n, 2 * C), _f32)] * n_rel,
    )


def _proj(x, kw, kb, qw, qb, vw, vb, ar, mr):
    n = x.shape[0]
    return _proj_fn(n, ar.shape[0])(x, kw, kb[None, :], qw, qb[None, :],
                                    vw, vb[None, :], ar, mr)


@functools.lru_cache(maxsize=None)
def _edge_fn(e_pad, e_real):
    te = 512
    grid = (e_pad // te,)
    inv_sqrt = 1.0 / (DH ** 0.5)

    def body(kavm_ref, q_ref, p_ref, msg0_out, msg1_out, ex_out):
        prod = kavm_ref[:, :C] * q_ref[...]
        rows = pl.program_id(0) * te + lax.broadcasted_iota(
            jnp.int32, (te, 1), 0)
        valid = rows < e_real
        exs = []
        for h in range(H):
            s = jnp.sum(prod[:, h * DH:(h + 1) * DH], axis=1, keepdims=True)
            e = jnp.exp(s * (p_ref[0, h] * inv_sqrt))
            e = jnp.where(valid, e, 0.0)
            exs.append(e)
            m = kavm_ref[:, C + h * DH:C + (h + 1) * DH] * e
            if h < 2:
                msg0_out[:, h * DH:(h + 1) * DH] = m
            else:
                msg1_out[:, (h - 2) * DH:(h - 1) * DH] = m
        ex_out[...] = jnp.concatenate(
            exs + [jnp.zeros((te, 4), _f32)], axis=1)

    return pl.pallas_call(
        body,
        grid=grid,
        in_specs=[
            pl.BlockSpec((te, 2 * C), lambda i: (i, 0)),
            pl.BlockSpec((te, C), lambda i: (i, 0)),
            pl.BlockSpec(memory_space=pltpu.SMEM),
        ],
        out_specs=[pl.BlockSpec((te, 128), lambda i: (i, 0)),
                   pl.BlockSpec((te, 128), lambda i: (i, 0)),
                   pl.BlockSpec((te, 8), lambda i: (i, 0))],
        out_shape=[_SDS((e_pad, 128), _f32), _SDS((e_pad, 128), _f32),
                   _SDS((e_pad, 8), _f32)],
    )


def _edge(kavm_sel, q_sel, p):
    e_pad = kavm_sel.shape[0]
    e_real = {EF_P: EF, EI_P: EI, EC_P: EC}[e_pad]
    return _edge_fn(e_pad, e_real)(kavm_sel, q_sel, p[None, :])


@functools.lru_cache(maxsize=None)
def _seg_tc_fn(e_pad):
    te = 512
    grid = (e_pad // te,)

    def body(dst_ref, msg0_ref, msg1_ref, ex_ref, num_out, den_out):
        i = pl.program_id(0)

        @pl.when(i == 0)
        def _():
            num_out[...] = jnp.zeros((N1_P, 2 * 128), _f32)
            den_out[...] = jnp.zeros((N1_P, 8), _f32)

        onehot = (dst_ref[...] == lax.broadcasted_iota(
            jnp.int32, (te, N1_P), 1)).astype(_f32)
        msg = jnp.concatenate([msg0_ref[...], msg1_ref[...]], axis=1)
        dn = (((0,), (0,)), ((), ()))
        num_out[...] += lax.dot_general(onehot, msg, dn,
                                        preferred_element_type=_f32)
        den_out[...] += lax.dot_general(onehot, ex_ref[...], dn,
                                        preferred_element_type=_f32)

    return pl.pallas_call(
        body,
        grid=grid,
        in_specs=[
            pl.BlockSpec((te, 1), lambda i: (i, 0)),
            pl.BlockSpec((te, 128), lambda i: (i, 0)),
            pl.BlockSpec((te, 128), lambda i: (i, 0)),
            pl.BlockSpec((te, 8), lambda i: (i, 0)),
        ],
        out_specs=[pl.BlockSpec((N1_P, 256), lambda i: (0, 0)),
                   pl.BlockSpec((N1_P, 8), lambda i: (0, 0))],
        out_shape=[_SDS((N1_P, 256), _f32), _SDS((N1_P, 8), _f32)],
    )


def _seg_tc(dst, msg0, msg1, ex):
    (e_pad,) = dst.shape
    return _seg_tc_fn(e_pad)(dst[:, None], msg0, msg1, ex)


@functools.lru_cache(maxsize=None)
def _post_fn(n, do_ln, den_parts):
    tn = 400
    grid = (n // tn,)

    def body(num_ref, den_ref, x_ref, aw_ref, ab_ref, sk_ref, g_ref, b_ref,
             out_ref):
        if den_parts:
            den = jnp.sum(den_ref[...], axis=0)
        else:
            den = den_ref[...]
        heads = [num_ref[:, h * DH:(h + 1) * DH] /
                 (den[:, h:h + 1] + 1e-16) for h in range(H)]
        agg = jnp.concatenate(heads, axis=1)
        o = jnp.dot(jax.nn.gelu(agg), aw_ref[...],
                    preferred_element_type=_f32) + ab_ref[...]
        sg = jax.nn.sigmoid(sk_ref[0])
        xn = sg * o + (1.0 - sg) * x_ref[...]
        if do_ln:
            mu = jnp.mean(xn, axis=1, keepdims=True)
            var = jnp.mean((xn - mu) ** 2, axis=1, keepdims=True)
            xn = (xn - mu) / jnp.sqrt(var + 1e-5) * g_ref[...] + b_ref[...]
        out_ref[...] = jax.nn.gelu(xn)

    full = lambda i: (0, 0)
    return pl.pallas_call(
        body,
        grid=grid,
        in_specs=[
            pl.BlockSpec((tn, C), lambda i: (i, 0)),
            (pl.BlockSpec((NS, tn, 8), lambda i: (0, i, 0)) if den_parts
             else pl.BlockSpec((tn, 8), lambda i: (i, 0))),
            pl.BlockSpec((tn, C), lambda i: (i, 0)),
            pl.BlockSpec((C, C), full), pl.BlockSpec((1, C), full),
            pl.BlockSpec(memory_space=pltpu.SMEM),
            pl.BlockSpec((1, C), full), pl.BlockSpec((1, C), full),
        ],
        out_specs=pl.BlockSpec((tn, C), lambda i: (i, 0)),
        out_shape=_SDS((n, C), _f32),
    )


def _post(num, den, x, aw, ab, sk, g, b, do_ln):
    n = x.shape[0]
    return _post_fn(n, do_ln, den.ndim == 3)(num, den, x, aw, ab[None, :],
                                             sk[None], g[None, :], b[None, :])


@functools.lru_cache(maxsize=None)
def _jk_fn(n):
    tn = 400
    grid = (n // tn,)

    def lstm_step(x, h, c, wi_ref, wh_ref, b_ref, first):
        z = jnp.dot(x, wi_ref[...], preferred_element_type=_f32) + b_ref[...]
        if not first:
            z = z + jnp.dot(h, wh_ref[...], preferred_element_type=_f32)
        zi = z[:, 0:C]
        zf = z[:, C:2 * C]
        zg = z[:, 2 * C:3 * C]
        zo = z[:, 3 * C:4 * C]
        gi = jax.nn.sigmoid(zi) * jnp.tanh(zg)
        c2 = gi if first else jax.nn.sigmoid(zf) * c + gi
        h2 = jax.nn.sigmoid(zo) * jnp.tanh(c2)
        return h2, c2

    def body(x1_ref, x2_ref, wif_ref, whf_ref, bf_ref, wib_ref, whb_ref,
             bb_ref, awt_ref, awb_ref, attb_ref, pw_ref, jk_out, sc_out):
        x1 = x1_ref[...]
        x2 = x2_ref[...]
        hf1, cf1 = lstm_step(x1, None, None, wif_ref, whf_ref, bf_ref, True)
        hf2, _ = lstm_step(x2, hf1, cf1, wif_ref, whf_ref, bf_ref, False)
        hb2, cb2 = lstm_step(x2, None, None, wib_ref, whb_ref, bb_ref, True)
        hb1, _ = lstm_step(x1, hb2, cb2, wib_ref, whb_ref, bb_ref, False)
        attb = attb_ref[0]
        a0 = (jnp.dot(hf1, awt_ref[...], preferred_element_type=_f32) +
              jnp.dot(hb1, awb_ref[...], preferred_element_type=_f32) + attb)
        a1 = (jnp.dot(hf2, awt_ref[...], preferred_element_type=_f32) +
              jnp.dot(hb2, awb_ref[...], preferred_element_type=_f32) + attb)
        m = jnp.maximum(a0, a1)
        e0 = jnp.exp(a0 - m)
        e1 = jnp.exp(a1 - m)
        tot = e0 + e1
        jk = (e0 / tot) * x1 + (e1 / tot) * x2
        jk_out[...] = jk
        sc_out[...] = jnp.dot(jk, pw_ref[...], preferred_element_type=_f32)

    full = lambda i: (0, 0)
    return pl.pallas_call(
        body,
        grid=grid,
        in_specs=[
            pl.BlockSpec((tn, C), lambda i: (i, 0)),
            pl.BlockSpec((tn, C), lambda i: (i, 0)),
            pl.BlockSpec((C, 4 * C), full), pl.BlockSpec((C, 4 * C), full),
            pl.BlockSpec((1, 4 * C), full),
            pl.BlockSpec((C, 4 * C), full), pl.BlockSpec((C, 4 * C), full),
            pl.BlockSpec((1, 4 * C), full),
            pl.BlockSpec((C, 1), full), pl.BlockSpec((C, 1), full),
            pl.BlockSpec(memory_space=pltpu.SMEM),
            pl.BlockSpec((C, 1), full),
        ],
        out_specs=[pl.BlockSpec((tn, C), lambda i: (i, 0)),
                   pl.BlockSpec((tn, 1), lambda i: (i, 0))],
        out_shape=[_SDS((n, C), _f32), _SDS((n, 1), _f32)],
    )


def _jk(x1, x2, wif, whf, bf, wib, whb, bb, attw, attb, pw):
    n = x1.shape[0]
    return _jk_fn(n)(x1, x2, wif, whf, bf[None, :], wib, whb, bb[None, :],
                     attw[:C], attw[C:], attb, pw[:, None])


NTOT_P = 10240


def _topk_mlp_body(s_ref, feats_ref, w1_ref, b1_ref, w2_ref, b2_ref,
                   w3_ref, b3_ref, w4_ref, b4_ref, out_ref):
    s = s_ref[...]
    idxm = (lax.broadcasted_iota(jnp.int32, s.shape, 0) * 128 +
            lax.broadcasted_iota(jnp.int32, s.shape, 1))
    acc = jnp.zeros((1, 1024), _f32)
    for j in range(8):
        m = jnp.max(s)
        cand = jnp.where(s == m, idxm, jnp.int32(2 ** 30))
        ix = jnp.min(cand)
        s = jnp.where(idxm == ix, jnp.float32(-3e38), s)
        row = feats_ref[pl.ds(ix, 1), :]
        acc = acc + jnp.dot(row * jnp.tanh(m), w1_ref[j],
                            preferred_element_type=_f32)
    h = jax.nn.gelu(acc + b1_ref[...])
    h = jax.nn.gelu(jnp.dot(h, w2_ref[...], preferred_element_type=_f32)
                    + b2_ref[...])
    h = jax.nn.gelu(jnp.dot(h, w3_ref[...], preferred_element_type=_f32)
                    + b3_ref[...])
    out_ref[...] = (jnp.dot(h, w4_ref[...], preferred_element_type=_f32)
                    + b4_ref[0])


@functools.lru_cache(maxsize=None)
def _topk_mlp_fn():
    return pl.pallas_call(
        _topk_mlp_body,
        in_specs=[
            pl.BlockSpec((NTOT_P // 128, 128), lambda: (0, 0)),
            pl.BlockSpec((NTOT_P, C), lambda: (0, 0)),
            pl.BlockSpec((8, C, 1024), lambda: (0, 0, 0)),
            pl.BlockSpec((1, 1024), lambda: (0, 0)),
            pl.BlockSpec((1024, 512), lambda: (0, 0)),
            pl.BlockSpec((1, 512), lambda: (0, 0)),
            pl.BlockSpec((512, C), lambda: (0, 0)),
            pl.BlockSpec((1, C), lambda: (0, 0)),
            pl.BlockSpec((C, 1), lambda: (0, 0)),
            pl.BlockSpec(memory_space=pltpu.SMEM),
        ],
        out_specs=pl.BlockSpec((1, 1), lambda: (0, 0)),
        out_shape=_SDS((1, 1), _f32),
    )


def _pad_idx(a, tot):
    return jnp.concatenate([a, jnp.zeros((tot - a.shape[0],), jnp.int32)])


def kernel(x_instr, x_block, ei_flow, ei_in, ei_cont, Kw, Kb, Qw, Qb, Vw, Vb,
           Aw, Ab, skip, a_rel, m_rel, p_rel, ln_g, ln_b, wi_f, wh_f, bl_f,
           wi_b, wh_b, bl_b, att_w, att_b, pool_w, pool_b, w1, b1, w2, b2,
           w3, b3, w4, b4):
    src_f = _pad_idx(ei_flow[0].astype(jnp.int32), EF_P)
    dst_f = _pad_idx(ei_flow[1].astype(jnp.int32), EF_P)
    src_i = _pad_idx(ei_in[0].astype(jnp.int32), EI_P)
    dst_i = _pad_idx(ei_in[1].astype(jnp.int32), EI_P)
    src_c = _pad_idx(ei_cont[0].astype(jnp.int32), EC_P)
    dst_c = _pad_idx(ei_cont[1].astype(jnp.int32), EC_P)

    xs = [x_instr, x_block]
    louts = []
    for l in range(2):
        q0, kavm_f, kavm_i = _proj(xs[0], Kw[l, 0], Kb[l, 0], Qw[l, 0],
                                   Qb[l, 0], Vw[l, 0], Vb[l, 0],
                                   a_rel[l, 0:2], m_rel[l, 0:2])
        q1, kavm_c = _proj(xs[1], Kw[l, 1], Kb[l, 1], Qw[l, 1], Qb[l, 1],
                           Vw[l, 1], Vb[l, 1], a_rel[l, 2:3], m_rel[l, 2:3])

        m0f, m1f, exf = _edge(_sc_gather(kavm_f, src_f),
                              _sc_gather(q0, dst_f), p_rel[l, 0])
        m0i, m1i, exi = _edge(_sc_gather(kavm_i, src_i),
                              _sc_gather(q1, dst_i), p_rel[l, 1])
        m0c, m1c, exc = _edge(_sc_gather(kavm_c, src_c),
                              _sc_gather(q0, dst_c), p_rel[l, 2])

        num00, num01, den0 = _sc_scatter(
            jnp.concatenate([dst_f, dst_c]),
            jnp.concatenate([m0f, m0c]),
            jnp.concatenate([m1f, m1c]),
            jnp.concatenate([exf, exc]).reshape(-1), N0_P)
        num1c, den1 = _seg_tc(dst_i, m0i, m1i, exi)

        num0 = jnp.concatenate([num00, num01], axis=1)[:N0]
        x0n = _post(num0, den0, xs[0], Aw[l, 0], Ab[l, 0], skip[l, 0],
                    ln_g[0], ln_b[0], l == 0)
        x1n = _post(num1c[:N1], den1[:N1], xs[1], Aw[l, 1], Ab[l, 1],
                    skip[l, 1], ln_g[1], ln_b[1], l == 0)
        xs = [x0n, x1n]
        louts.append(xs)

    jk0, sc0 = _jk(louts[0][0], louts[1][0], wi_f[0], wh_f[0], bl_f[0],
                   wi_b[0], wh_b[0], bl_b[0], att_w[0], att_b[0], pool_w[0])
    jk1, sc1 = _jk(louts[0][1], louts[1][1], wi_f[1], wh_f[1], bl_f[1],
                   wi_b[1], wh_b[1], bl_b[1], att_w[1], att_b[1], pool_w[1])

    scores = jnp.concatenate([sc0[:, 0] + pool_b[0], sc1[:, 0] + pool_b[1],
                              jnp.full((NTOT_P - N0 - N1,), -1e30, _f32)])
    feats = jnp.concatenate(
        [jk0, jk1, jnp.zeros((NTOT_P - N0 - N1, C), _f32)], axis=0)

    out = _topk_mlp_fn()(
        scores.reshape(NTOT_P // 128, 128), feats,
        w1.reshape(8, C, 1024), b1[None, :], w2, b2[None, :], w3,
        b3[None, :], w4, b4)
    return out
